# baseline (speedup 1.0000x reference)
import functools

import jax
import jax.numpy as jnp
from jax import lax
from jax.experimental import pallas as pl
from jax.experimental.pallas import tpu as pltpu
from jax.experimental.pallas import tpu_sc as plsc

K = 20
B, N, D = 4, 2048, 128
CO = 256
R = 1024
RF = 1024

_NC, _NS, _L = 2, 16, 16
_NW = _NC * _NS
_NPW = N // _NW
_G = 16
_NG = _NPW // _G
_P = 4
_NPK = _NPW // _P
_PR = _P * K


def _knn_proj_body(xr_ref, xb_ref, w_ref, ia_ref, ib_ref, a_ref, bm_ref):
    xr = xr_ref[...]
    xb = xb_ref[...]
    w1 = w_ref[:, :D]
    w2 = w_ref[:, D:]
    dn = (((1,), (1,)), ((), ()))
    a_ref[...] = lax.dot_general(xr, w1 - w2, dn, preferred_element_type=jnp.float32)
    bm_ref[...] = lax.dot_general(xr, w2, dn, preferred_element_type=jnp.float32)
    prod = lax.dot_general(xr, xb, dn, preferred_element_type=jnp.float32)
    ones = jnp.ones((1, D), jnp.float32)
    xxb = lax.dot_general(ones, xb * xb, dn, preferred_element_type=jnp.float32,
                          precision=lax.Precision.HIGHEST)
    xxr = jnp.sum(xr * xr, axis=1, keepdims=True)
    d = (xxr + (-2.0) * prod) + xxb
    iotaf = lax.broadcasted_iota(jnp.int32, (R, N), 1).astype(jnp.float32)
    cols = []
    for j in range(K + 1):
        m = jnp.min(d, axis=1, keepdims=True)
        cand = jnp.where(d == m, iotaf, jnp.float32(N))
        am = jnp.min(cand, axis=1, keepdims=True)
        if j > 0:
            cols.append(am)
        if j < K:
            d = jnp.where(iotaf == am, jnp.float32(jnp.inf), d)
    idxg = jnp.concatenate(cols, axis=1).astype(jnp.int32)
    ia_ref[...] = idxg * 2
    ib_ref[...] = idxg * 2 + 1


def _knn_proj(xb2, W):
    return pl.pallas_call(
        _knn_proj_body,
        grid=(N // R,),
        in_specs=[
            pl.BlockSpec((R, D), lambda r: (r, 0)),
            pl.BlockSpec((N, D), lambda r: (0, 0)),
            pl.BlockSpec((CO, 2 * D), lambda r: (0, 0)),
        ],
        out_specs=[
            pl.BlockSpec((R, K), lambda r: (r, 0)),
            pl.BlockSpec((R, K), lambda r: (r, 0)),
            pl.BlockSpec((R, CO), lambda r: (r, 0)),
            pl.BlockSpec((R, CO), lambda r: (r, 0)),
        ],
        out_shape=[
            jax.ShapeDtypeStruct((N, K), jnp.int32),
            jax.ShapeDtypeStruct((N, K), jnp.int32),
            jax.ShapeDtypeStruct((N, CO), jnp.float32),
            jax.ShapeDtypeStruct((N, CO), jnp.float32),
        ],
    )(xb2, xb2, W)


def _sc_body(bm_hbm, ia_hbm, ib_hbm, a_hbm, hmax_hbm, hmin_hbm, ps_hbm, pss_hbm,
             ia_v, ib_v, a_v, ga0, gb0, ga1, gb1, hx_v, hn_v, ps_v, pss_v,
             semA, semB):
    wid = lax.axis_index("s") * _NC + lax.axis_index("c")
    base = wid * _NPW
    pltpu.sync_copy(ia_hbm.at[pl.ds(wid * _NPK, _NPK), :], ia_v)
    pltpu.sync_copy(ib_hbm.at[pl.ds(wid * _NPK, _NPK), :], ib_v)
    for h in range(2):
        for c in range(8):
            sl = pl.ds(c * _L, _L)
            ps_v[h, sl] = jnp.zeros((_L,), jnp.float32)
            pss_v[h, sl] = jnp.zeros((_L,), jnp.float32)

    bufs = ((ga0, gb0, semA), (ga1, gb1, semB))

    def fire(p, par):
        gx, gy, sem = bufs[par]
        pltpu.async_copy(bm_hbm.at[ia_v.at[p]], gx, sem)
        pltpu.async_copy(bm_hbm.at[ib_v.at[p]], gy, sem)

    def drain(par):
        gx, gy, sem = bufs[par]
        pltpu.make_async_copy(bm_hbm.at[pl.ds(0, _PR), :], gx, sem).wait()
        pltpu.make_async_copy(bm_hbm.at[pl.ds(0, _PR), :], gy, sem).wait()

    def process(pk, par):
        gx, gy, sem = bufs[par]

        def node_body(q, carry):
            i_loc = 4 * pk + q
            r0 = K * q
            for h, gh in ((0, gx), (1, gy)):
                for c in range(8):
                    sl = pl.ds(c * _L, _L)
                    v0 = gh[r0, sl]
                    mx, mn, s, ss = v0, v0, v0, v0 * v0
                    for j in range(1, K):
                        v = gh[r0 + j, sl]
                        mx = jnp.maximum(mx, v)
                        mn = jnp.minimum(mn, v)
                        s = s + v
                        ss = ss + v * v
                    a = a_v[2 * i_loc + h, sl]
                    ps_v[h, sl] = ps_v[h, sl] + (jnp.float32(K) * a + s)
                    pss_v[h, sl] = pss_v[h, sl] + (
                        jnp.float32(K) * (a * a) + 2.0 * (a * s) + ss)
                    hx_v[2 * i_loc + h, sl] = a + mx
                    hn_v[2 * i_loc + h, sl] = a + mn
            return carry

        lax.fori_loop(0, _P, node_body, 0)

    fire(0, 0)

    def group_body(g, carry):
        pltpu.sync_copy(a_hbm.at[pl.ds(2 * base + 32 * g, 32), :], a_v)
        for pk in range(4):
            p = 4 * g + pk
            par = pk & 1
            if pk < 3:
                fire(p + 1, 1 - par)
            else:
                @pl.when(g < _NG - 1)
                def _():
                    fire(p + 1, 1 - par)
            drain(par)
            process(pk, par)
        pltpu.sync_copy(hx_v, hmax_hbm.at[pl.ds(2 * base + 32 * g, 32), :])
        pltpu.sync_copy(hn_v, hmin_hbm.at[pl.ds(2 * base + 32 * g, 32), :])
        return carry

    lax.fori_loop(0, _NG, group_body, 0)
    pltpu.sync_copy(ps_v, ps_hbm.at[pl.ds(2 * wid, 2), :])
    pltpu.sync_copy(pss_v, pss_hbm.at[pl.ds(2 * wid, 2), :])


_sc_gather_reduce = functools.partial(
    pl.kernel,
    mesh=plsc.VectorSubcoreMesh(core_axis_name="c", subcore_axis_name="s"),
    out_type=[
        jax.ShapeDtypeStruct((2 * N, 128), jnp.float32),
        jax.ShapeDtypeStruct((2 * N, 128), jnp.float32),
        jax.ShapeDtypeStruct((2 * _NW, 128), jnp.float32),
        jax.ShapeDtypeStruct((2 * _NW, 128), jnp.float32),
    ],
    scratch_types=[
        pltpu.VMEM((_NPK, _PR), jnp.int32),
        pltpu.VMEM((_NPK, _PR), jnp.int32),
        pltpu.VMEM((2 * _G, 128), jnp.float32),
        pltpu.VMEM((_PR, 128), jnp.float32),
        pltpu.VMEM((_PR, 128), jnp.float32),
        pltpu.VMEM((_PR, 128), jnp.float32),
        pltpu.VMEM((_PR, 128), jnp.float32),
        pltpu.VMEM((2 * _G, 128), jnp.float32),
        pltpu.VMEM((2 * _G, 128), jnp.float32),
        pltpu.VMEM((2, 128), jnp.float32),
        pltpu.VMEM((2, 128), jnp.float32),
        pltpu.SemaphoreType.DMA,
        pltpu.SemaphoreType.DMA,
    ],
)(_sc_body)


def _final_body(hx_ref, hn_ref, ps_ref, pss_ref, g_ref, b_ref, o_ref):
    par = lax.broadcasted_iota(jnp.int32, (2 * B * _NW, 1), 0) % 2
    ps = ps_ref[...]
    pss = pss_ref[...]
    cnt = jnp.float32(B * N * K)
    tot0 = jnp.sum(jnp.where(par == 0, ps, 0.0), axis=0, keepdims=True)
    tot1 = jnp.sum(jnp.where(par == 1, ps, 0.0), axis=0, keepdims=True)
    tss0 = jnp.sum(jnp.where(par == 0, pss, 0.0), axis=0, keepdims=True)
    tss1 = jnp.sum(jnp.where(par == 1, pss, 0.0), axis=0, keepdims=True)
    mean0, mean1 = tot0 / cnt, tot1 / cnt
    var0 = tss0 / cnt - mean0 * mean0
    var1 = tss1 / cnt - mean1 * mean1
    sc0 = g_ref[0:1, :] * lax.rsqrt(var0 + 1e-5)
    sc1 = g_ref[1:2, :] * lax.rsqrt(var1 + 1e-5)
    bi0 = b_ref[0:1, :] - mean0 * sc0
    bi1 = b_ref[1:2, :] - mean1 * sc1
    rpar = lax.broadcasted_iota(jnp.int32, (RF, 1), 0) % 2
    scale = jnp.where(rpar == 0, sc0, sc1)
    bias = jnp.where(rpar == 0, bi0, bi1)

    def act(t):
        u = t * scale + bias
        return jnp.where(u >= 0, u, 0.2 * u)

    o_ref[...] = jnp.maximum(act(hx_ref[...]), act(hn_ref[...]))


def _final(hmax, hmin, ps, pss, gamma, beta):
    return pl.pallas_call(
        _final_body,
        grid=(2 * B * N // RF,),
        in_specs=[
            pl.BlockSpec((RF, 128), lambda i: (i, 0)),
            pl.BlockSpec((RF, 128), lambda i: (i, 0)),
            pl.BlockSpec((2 * B * _NW, 128), lambda i: (0, 0)),
            pl.BlockSpec((2 * B * _NW, 128), lambda i: (0, 0)),
            pl.BlockSpec((2, 128), lambda i: (0, 0)),
            pl.BlockSpec((2, 128), lambda i: (0, 0)),
        ],
        out_specs=pl.BlockSpec((RF, 128), lambda i: (i, 0)),
        out_shape=jax.ShapeDtypeStruct((2 * B * N, 128), jnp.float32),
    )(hmax, hmin, ps, pss, gamma.reshape(2, 128), beta.reshape(2, 128))


def kernel(x, W, gamma, beta):
    hx_l, hn_l, ps_l, pss_l = [], [], [], []
    for b in range(B):
        ia, ib, a2, bm2 = _knn_proj(x[b], W)
        hx, hn, ps, pss = _sc_gather_reduce(
            bm2.reshape(2 * N, 128),
            ia.reshape(N // _P, _PR),
            ib.reshape(N // _P, _PR),
            a2.reshape(2 * N, 128))
        hx_l.append(hx)
        hn_l.append(hn)
        ps_l.append(ps)
        pss_l.append(pss)
    hmax = jnp.concatenate(hx_l, axis=0)
    hmin = jnp.concatenate(hn_l, axis=0)
    ps = jnp.concatenate(ps_l, axis=0)
    pss = jnp.concatenate(pss_l, axis=0)
    outf = _final(hmax, hmin, ps, pss, gamma, beta)
    return outf.reshape(B, N, CO)


# speedup vs baseline: 1.2967x; 1.2967x over previous
import functools

import jax
import jax.numpy as jnp
from jax import lax
from jax.experimental import pallas as pl
from jax.experimental.pallas import tpu as pltpu
from jax.experimental.pallas import tpu_sc as plsc

K = 20
B, N, D = 4, 2048, 128
CO = 256
R = 512
RF = 1024

_NC, _NS, _L = 2, 16, 16
_NW = _NC * _NS
_NPW = N // _NW
_G = 16
_NG = _NPW // _G
_P = 4
_NPK = _NPW // _P
_PR = _P * K


def _knn_proj_body(xr_ref, xb_ref, w_ref, ia_ref, ib_ref, a_ref, bm_ref):
    xr = xr_ref[...]
    xb = xb_ref[...]
    w1 = w_ref[:, :D]
    w2 = w_ref[:, D:]
    dn = (((1,), (1,)), ((), ()))
    a_ref[...] = lax.dot_general(xr, w1 - w2, dn, preferred_element_type=jnp.float32)
    bm_ref[...] = lax.dot_general(xr, w2, dn, preferred_element_type=jnp.float32)
    prod = lax.dot_general(xr, xb, dn, preferred_element_type=jnp.float32)
    ones = jnp.ones((1, D), jnp.float32)
    xxb = lax.dot_general(ones, xb * xb, dn, preferred_element_type=jnp.float32,
                          precision=lax.Precision.HIGHEST)
    xxr = jnp.sum(xr * xr, axis=1, keepdims=True)
    d = (xxr + (-2.0) * prod) + xxb
    iotaf = lax.broadcasted_iota(jnp.int32, (R, N), 1).astype(jnp.float32)
    cols = []
    for j in range(K + 1):
        m = jnp.min(d, axis=1, keepdims=True)
        cand = jnp.where(d == m, iotaf, jnp.float32(N))
        am = jnp.min(cand, axis=1, keepdims=True)
        if j > 0:
            cols.append(am)
        if j < K:
            d = jnp.where(iotaf == am, jnp.float32(jnp.inf), d)
    idxg = jnp.concatenate(cols, axis=1).astype(jnp.int32)
    ia_ref[...] = idxg * 2
    ib_ref[...] = idxg * 2 + 1


def _knn_proj(xb2, W):
    return pl.pallas_call(
        _knn_proj_body,
        grid=(N // R,),
        in_specs=[
            pl.BlockSpec((R, D), lambda r: (r, 0)),
            pl.BlockSpec((N, D), lambda r: (0, 0)),
            pl.BlockSpec((CO, 2 * D), lambda r: (0, 0)),
        ],
        out_specs=[
            pl.BlockSpec((R, K), lambda r: (r, 0)),
            pl.BlockSpec((R, K), lambda r: (r, 0)),
            pl.BlockSpec((R, CO), lambda r: (r, 0)),
            pl.BlockSpec((R, CO), lambda r: (r, 0)),
        ],
        out_shape=[
            jax.ShapeDtypeStruct((N, K), jnp.int32),
            jax.ShapeDtypeStruct((N, K), jnp.int32),
            jax.ShapeDtypeStruct((N, CO), jnp.float32),
            jax.ShapeDtypeStruct((N, CO), jnp.float32),
        ],
    )(xb2, xb2, W)


def _sc_body(bm_hbm, ia_hbm, ib_hbm, a_hbm, hmax_hbm, hmin_hbm, ps_hbm, pss_hbm,
             ia_v, ib_v, a_v, ga0, gb0, ga1, gb1, hx_v, hn_v, ps_v, pss_v,
             semA, semB):
    wid = lax.axis_index("s") * _NC + lax.axis_index("c")
    base = wid * _NPW
    pltpu.sync_copy(ia_hbm.at[pl.ds(wid * _NPK, _NPK), :], ia_v)
    pltpu.sync_copy(ib_hbm.at[pl.ds(wid * _NPK, _NPK), :], ib_v)
    for h in range(2):
        for c in range(8):
            sl = pl.ds(c * _L, _L)
            ps_v[h, sl] = jnp.zeros((_L,), jnp.float32)
            pss_v[h, sl] = jnp.zeros((_L,), jnp.float32)

    bufs = ((ga0, gb0, semA), (ga1, gb1, semB))

    def fire(p, par):
        gx, gy, sem = bufs[par]
        pltpu.async_copy(bm_hbm.at[ia_v.at[p]], gx, sem)
        pltpu.async_copy(bm_hbm.at[ib_v.at[p]], gy, sem)

    def drain(par):
        gx, gy, sem = bufs[par]
        pltpu.make_async_copy(bm_hbm.at[pl.ds(0, _PR), :], gx, sem).wait()
        pltpu.make_async_copy(bm_hbm.at[pl.ds(0, _PR), :], gy, sem).wait()

    def process(pk, par):
        gx, gy, sem = bufs[par]

        def node_body(q, carry):
            i_loc = 4 * pk + q
            r0 = K * q
            for h, gh in ((0, gx), (1, gy)):
                for c in range(8):
                    sl = pl.ds(c * _L, _L)
                    v0 = gh[r0, sl]
                    mx, mn, s, ss = v0, v0, v0, v0 * v0
                    for j in range(1, K):
                        v = gh[r0 + j, sl]
                        mx = jnp.maximum(mx, v)
                        mn = jnp.minimum(mn, v)
                        s = s + v
                        ss = ss + v * v
                    a = a_v[2 * i_loc + h, sl]
                    ps_v[h, sl] = ps_v[h, sl] + (jnp.float32(K) * a + s)
                    pss_v[h, sl] = pss_v[h, sl] + (
                        jnp.float32(K) * (a * a) + 2.0 * (a * s) + ss)
                    hx_v[2 * i_loc + h, sl] = a + mx
                    hn_v[2 * i_loc + h, sl] = a + mn
            return carry

        lax.fori_loop(0, _P, node_body, 0)

    fire(0, 0)

    def group_body(g, carry):
        pltpu.sync_copy(a_hbm.at[pl.ds(2 * base + 32 * g, 32), :], a_v)
        for pk in range(4):
            p = 4 * g + pk
            par = pk & 1
            if pk < 3:
                fire(p + 1, 1 - par)
            else:
                @pl.when(g < _NG - 1)
                def _():
                    fire(p + 1, 1 - par)
            drain(par)
            process(pk, par)
        pltpu.sync_copy(hx_v, hmax_hbm.at[pl.ds(2 * base + 32 * g, 32), :])
        pltpu.sync_copy(hn_v, hmin_hbm.at[pl.ds(2 * base + 32 * g, 32), :])
        return carry

    lax.fori_loop(0, _NG, group_body, 0)
    pltpu.sync_copy(ps_v, ps_hbm.at[pl.ds(2 * wid, 2), :])
    pltpu.sync_copy(pss_v, pss_hbm.at[pl.ds(2 * wid, 2), :])


_sc_gather_reduce = functools.partial(
    pl.kernel,
    mesh=plsc.VectorSubcoreMesh(core_axis_name="c", subcore_axis_name="s"),
    out_type=[
        jax.ShapeDtypeStruct((2 * N, 128), jnp.float32),
        jax.ShapeDtypeStruct((2 * N, 128), jnp.float32),
        jax.ShapeDtypeStruct((2 * _NW, 128), jnp.float32),
        jax.ShapeDtypeStruct((2 * _NW, 128), jnp.float32),
    ],
    scratch_types=[
        pltpu.VMEM((_NPK, _PR), jnp.int32),
        pltpu.VMEM((_NPK, _PR), jnp.int32),
        pltpu.VMEM((2 * _G, 128), jnp.float32),
        pltpu.VMEM((_PR, 128), jnp.float32),
        pltpu.VMEM((_PR, 128), jnp.float32),
        pltpu.VMEM((_PR, 128), jnp.float32),
        pltpu.VMEM((_PR, 128), jnp.float32),
        pltpu.VMEM((2 * _G, 128), jnp.float32),
        pltpu.VMEM((2 * _G, 128), jnp.float32),
        pltpu.VMEM((2, 128), jnp.float32),
        pltpu.VMEM((2, 128), jnp.float32),
        pltpu.SemaphoreType.DMA,
        pltpu.SemaphoreType.DMA,
    ],
)(_sc_body)


def _final_body(hx_ref, hn_ref, ps_ref, pss_ref, g_ref, b_ref, o_ref):
    par = lax.broadcasted_iota(jnp.int32, (2 * B * _NW, 1), 0) % 2
    ps = ps_ref[...]
    pss = pss_ref[...]
    cnt = jnp.float32(B * N * K)
    tot0 = jnp.sum(jnp.where(par == 0, ps, 0.0), axis=0, keepdims=True)
    tot1 = jnp.sum(jnp.where(par == 1, ps, 0.0), axis=0, keepdims=True)
    tss0 = jnp.sum(jnp.where(par == 0, pss, 0.0), axis=0, keepdims=True)
    tss1 = jnp.sum(jnp.where(par == 1, pss, 0.0), axis=0, keepdims=True)
    mean0, mean1 = tot0 / cnt, tot1 / cnt
    var0 = tss0 / cnt - mean0 * mean0
    var1 = tss1 / cnt - mean1 * mean1
    sc0 = g_ref[0:1, :] * lax.rsqrt(var0 + 1e-5)
    sc1 = g_ref[1:2, :] * lax.rsqrt(var1 + 1e-5)
    bi0 = b_ref[0:1, :] - mean0 * sc0
    bi1 = b_ref[1:2, :] - mean1 * sc1
    rpar = lax.broadcasted_iota(jnp.int32, (RF, 1), 0) % 2
    scale = jnp.where(rpar == 0, sc0, sc1)
    bias = jnp.where(rpar == 0, bi0, bi1)

    def act(t):
        u = t * scale + bias
        return jnp.where(u >= 0, u, 0.2 * u)

    o_ref[...] = jnp.maximum(act(hx_ref[...]), act(hn_ref[...]))


def _final(hmax, hmin, ps, pss, gamma, beta):
    return pl.pallas_call(
        _final_body,
        grid=(2 * B * N // RF,),
        in_specs=[
            pl.BlockSpec((RF, 128), lambda i: (i, 0)),
            pl.BlockSpec((RF, 128), lambda i: (i, 0)),
            pl.BlockSpec((2 * B * _NW, 128), lambda i: (0, 0)),
            pl.BlockSpec((2 * B * _NW, 128), lambda i: (0, 0)),
            pl.BlockSpec((2, 128), lambda i: (0, 0)),
            pl.BlockSpec((2, 128), lambda i: (0, 0)),
        ],
        out_specs=pl.BlockSpec((RF, 128), lambda i: (i, 0)),
        out_shape=jax.ShapeDtypeStruct((2 * B * N, 128), jnp.float32),
    )(hmax, hmin, ps, pss, gamma.reshape(2, 128), beta.reshape(2, 128))


def kernel(x, W, gamma, beta):
    hx_l, hn_l, ps_l, pss_l = [], [], [], []
    for b in range(B):
        ia, ib, a2, bm2 = _knn_proj(x[b], W)
        hx, hn, ps, pss = _sc_gather_reduce(
            bm2.reshape(2 * N, 128),
            ia.reshape(N // _P, _PR),
            ib.reshape(N // _P, _PR),
            a2.reshape(2 * N, 128))
        hx_l.append(hx)
        hn_l.append(hn)
        ps_l.append(ps)
        pss_l.append(pss)
    hmax = jnp.concatenate(hx_l, axis=0)
    hmin = jnp.concatenate(hn_l, axis=0)
    ps = jnp.concatenate(ps_l, axis=0)
    pss = jnp.concatenate(pss_l, axis=0)
    outf = _final(hmax, hmin, ps, pss, gamma, beta)
    return outf.reshape(B, N, CO)
